# baseline (speedup 1.0000x reference)
import functools

import jax
import jax.numpy as jnp
from jax import lax
from jax.experimental import pallas as pl
from jax.experimental.pallas import tpu as pltpu
from jax.experimental.pallas import tpu_sc as plsc


_BM = 200
_NBUF = 4


def _mm_body(s_hbm, x_ref, w_ref, o_ref, bufs, sems):
    nsteps = s_hbm.shape[0] // _BM

    def _copy(j, slot):
        return pltpu.make_async_copy(
            s_hbm.at[pl.ds(j * _BM, _BM)], bufs.at[slot], sems.at[slot])

    for j in range(_NBUF - 1):
        _copy(j, j).start()
    for j in range(nsteps):
        slot = j % _NBUF
        nxt = j + _NBUF - 1
        if nxt < nsteps:
            _copy(nxt, nxt % _NBUF).start()
        _copy(j, slot).wait()
        sx = jnp.dot(bufs[slot], x_ref[...],
                     preferred_element_type=jnp.float32)
        o_ref[pl.ds(j * _BM, _BM), :] = jnp.dot(
            sx, w_ref[...], preferred_element_type=jnp.float32)


def _spmm(sup, x, w):
    n, k = sup.shape
    d = w.shape[1]
    return pl.pallas_call(
        _mm_body,
        in_specs=[
            pl.BlockSpec(memory_space=pltpu.HBM),
            pl.BlockSpec((k, x.shape[1]), lambda: (0, 0)),
            pl.BlockSpec(w.shape, lambda: (0, 0)),
        ],
        out_specs=pl.BlockSpec((n, d), lambda: (0, 0)),
        out_shape=jax.ShapeDtypeStruct((n, d), jnp.float32),
        scratch_shapes=[
            pltpu.VMEM((_NBUF, _BM, k), jnp.float32),
            pltpu.SemaphoreType.DMA((_NBUF,)),
        ],
    )(sup, x, w)


_NW = 32
_CHUNK = 80
_CPW = 4


def _make_gather(n, d):
    mesh = plsc.VectorSubcoreMesh(core_axis_name="c", subcore_axis_name="s")
    n_chunks = n // _CHUNK

    @functools.partial(
        pl.kernel,
        mesh=mesh,
        out_type=jax.ShapeDtypeStruct((n, d), jnp.float32),
        scratch_types=[
            pltpu.VMEM((_CPW, _CHUNK), jnp.int32),
            pltpu.VMEM((_CPW, _CHUNK, d), jnp.float32),
            pltpu.SemaphoreType.DMA,
            pltpu.SemaphoreType.DMA,
        ],
    )
    def gk(table_hbm, idx_hbm, out_hbm, idx_v, rows_v, gsem, wsem):
        wid = lax.axis_index("s") * 2 + lax.axis_index("c")
        base_chunk = wid * _CPW
        tail_wid = n_chunks // _CPW
        tail_count = n_chunks % _CPW

        @pl.when(wid < tail_wid)
        def _full():
            pltpu.sync_copy(idx_hbm.at[pl.ds(base_chunk, _CPW)], idx_v)
            gathers = [
                pltpu.async_copy(table_hbm.at[idx_v.at[c]], rows_v.at[c], gsem)
                for c in range(_CPW)
            ]
            writes = []
            for c in range(_CPW):
                gathers[c].wait()
                writes.append(pltpu.async_copy(
                    rows_v.at[c],
                    out_hbm.at[pl.ds((base_chunk + c) * _CHUNK, _CHUNK)],
                    wsem))
            for wr in writes:
                wr.wait()

        @pl.when(wid == tail_wid)
        def _tail():
            for c in range(tail_count):
                pltpu.sync_copy(idx_hbm.at[pl.ds(base_chunk + c, 1)],
                                idx_v.at[pl.ds(c, 1)])
                pltpu.async_copy(
                    table_hbm.at[idx_v.at[c]], rows_v.at[c], gsem).wait()

                pltpu.sync_copy(
                    rows_v.at[c],
                    out_hbm.at[pl.ds((base_chunk + c) * _CHUNK, _CHUNK)])

    return gk


def kernel(x, selected_index, support, w):
    n = x.shape[0]
    s = _spmm(support[0], x, w)
    for i in range(1, support.shape[0]):
        s = s + _spmm(support[i], x, w)

    n_chunks = n // _CHUNK
    pad_chunks = -(-n_chunks // _CPW) * _CPW - n_chunks
    idx2d = jnp.concatenate([
        selected_index.astype(jnp.int32).reshape(n_chunks, _CHUNK),
        jnp.zeros((pad_chunks, _CHUNK), jnp.int32)])
    return _make_gather(n, w.shape[1])(s, idx2d)


# speedup vs baseline: 1.0071x; 1.0071x over previous
import functools

import jax
import jax.numpy as jnp
from jax import lax
from jax.experimental import pallas as pl
from jax.experimental.pallas import tpu as pltpu
from jax.experimental.pallas import tpu_sc as plsc


_BM = 400
_NBUF = 3


def _mm_body(s_hbm, x_ref, w_ref, o_ref, bufs, sems):
    nsteps = s_hbm.shape[0] // _BM

    def _copy(j, slot):
        return pltpu.make_async_copy(
            s_hbm.at[pl.ds(j * _BM, _BM)], bufs.at[slot], sems.at[slot])

    for j in range(_NBUF - 1):
        _copy(j, j).start()
    for j in range(nsteps):
        slot = j % _NBUF
        nxt = j + _NBUF - 1
        if nxt < nsteps:
            _copy(nxt, nxt % _NBUF).start()
        _copy(j, slot).wait()
        sx = jnp.dot(bufs[slot], x_ref[...],
                     preferred_element_type=jnp.float32)
        o_ref[pl.ds(j * _BM, _BM), :] = jnp.dot(
            sx, w_ref[...], preferred_element_type=jnp.float32)


def _spmm(sup, x, w):
    n, k = sup.shape
    d = w.shape[1]
    return pl.pallas_call(
        _mm_body,
        in_specs=[
            pl.BlockSpec(memory_space=pltpu.HBM),
            pl.BlockSpec((k, x.shape[1]), lambda: (0, 0)),
            pl.BlockSpec(w.shape, lambda: (0, 0)),
        ],
        out_specs=pl.BlockSpec((n, d), lambda: (0, 0)),
        out_shape=jax.ShapeDtypeStruct((n, d), jnp.float32),
        scratch_shapes=[
            pltpu.VMEM((_NBUF, _BM, k), jnp.float32),
            pltpu.SemaphoreType.DMA((_NBUF,)),
        ],
    )(sup, x, w)


_NW = 32
_CHUNK = 80
_CPW = 4


def _make_gather(n, d):
    mesh = plsc.VectorSubcoreMesh(core_axis_name="c", subcore_axis_name="s")
    n_chunks = n // _CHUNK

    @functools.partial(
        pl.kernel,
        mesh=mesh,
        out_type=jax.ShapeDtypeStruct((n, d), jnp.float32),
        scratch_types=[
            pltpu.VMEM((_CPW, _CHUNK), jnp.int32),
            pltpu.VMEM((_CPW, _CHUNK, d), jnp.float32),
            pltpu.SemaphoreType.DMA,
            pltpu.SemaphoreType.DMA,
        ],
    )
    def gk(table_hbm, idx_hbm, out_hbm, idx_v, rows_v, gsem, wsem):
        wid = lax.axis_index("s") * 2 + lax.axis_index("c")
        base_chunk = wid * _CPW
        tail_wid = n_chunks // _CPW
        tail_count = n_chunks % _CPW

        @pl.when(wid < tail_wid)
        def _full():
            pltpu.sync_copy(idx_hbm.at[pl.ds(base_chunk, _CPW)], idx_v)
            gathers = [
                pltpu.async_copy(table_hbm.at[idx_v.at[c]], rows_v.at[c], gsem)
                for c in range(_CPW)
            ]
            writes = []
            for c in range(_CPW):
                gathers[c].wait()
                writes.append(pltpu.async_copy(
                    rows_v.at[c],
                    out_hbm.at[pl.ds((base_chunk + c) * _CHUNK, _CHUNK)],
                    wsem))
            for wr in writes:
                wr.wait()

        @pl.when(wid == tail_wid)
        def _tail():
            for c in range(tail_count):
                pltpu.sync_copy(idx_hbm.at[pl.ds(base_chunk + c, 1)],
                                idx_v.at[pl.ds(c, 1)])
                pltpu.async_copy(
                    table_hbm.at[idx_v.at[c]], rows_v.at[c], gsem).wait()

                pltpu.sync_copy(
                    rows_v.at[c],
                    out_hbm.at[pl.ds((base_chunk + c) * _CHUNK, _CHUNK)])

    return gk


def kernel(x, selected_index, support, w):
    n = x.shape[0]
    s = _spmm(support[0], x, w)
    for i in range(1, support.shape[0]):
        s = s + _spmm(support[i], x, w)

    n_chunks = n // _CHUNK
    pad_chunks = -(-n_chunks // _CPW) * _CPW - n_chunks
    idx2d = jnp.concatenate([
        selected_index.astype(jnp.int32).reshape(n_chunks, _CHUNK),
        jnp.zeros((pad_chunks, _CHUNK), jnp.int32)])
    return _make_gather(n, w.shape[1])(s, idx2d)


# speedup vs baseline: 1.0493x; 1.0419x over previous
import functools

import jax
import jax.numpy as jnp
from jax import lax
from jax.experimental import pallas as pl
from jax.experimental.pallas import tpu as pltpu
from jax.experimental.pallas import tpu_sc as plsc


_BM = 400


def _mm_body(s_ref, x_ref, w_ref, o_ref):
    sx = jnp.dot(s_ref[...], x_ref[...], preferred_element_type=jnp.float32)
    o_ref[...] = jnp.dot(sx, w_ref[...], preferred_element_type=jnp.float32)


def _spmm(sup, x, w):
    n, k = sup.shape
    d = w.shape[1]
    return pl.pallas_call(
        _mm_body,
        grid=(n // _BM,),
        in_specs=[
            pl.BlockSpec((_BM, k), lambda i: (i, 0)),
            pl.BlockSpec((k, x.shape[1]), lambda i: (0, 0)),
            pl.BlockSpec(w.shape, lambda i: (0, 0)),
        ],
        out_specs=pl.BlockSpec((_BM, d), lambda i: (i, 0)),
        out_shape=jax.ShapeDtypeStruct((n, d), jnp.float32),
    )(sup, x, w)


_NW = 32
_CHUNK = 80
_CPW = 4


def _make_gather(n, d):
    mesh = plsc.VectorSubcoreMesh(core_axis_name="c", subcore_axis_name="s")
    n_chunks = n // _CHUNK
    per_w = _CPW * _CHUNK
    tail_wid = n_chunks // _CPW
    tail_count = n_chunks % _CPW

    @functools.partial(
        pl.kernel,
        mesh=mesh,
        out_type=jax.ShapeDtypeStruct((n, d), jnp.float32),
        scratch_types=[
            pltpu.VMEM((per_w,), jnp.int32),
            pltpu.VMEM((_CPW, _CHUNK, d), jnp.float32),
            pltpu.SemaphoreType.DMA,
            pltpu.SemaphoreType.DMA,
        ],
    )
    def gk(table_hbm, idx_hbm, out_hbm, idx_v, rows_v, gsem, wsem):
        wid = lax.axis_index("s") * 2 + lax.axis_index("c")
        base = wid * per_w

        @pl.when(wid < tail_wid)
        def _full():
            pltpu.sync_copy(idx_hbm.at[pl.ds(base, per_w)], idx_v)
            gathers = [
                pltpu.async_copy(
                    table_hbm.at[idx_v.at[pl.ds(c * _CHUNK, _CHUNK)]],
                    rows_v.at[c], gsem)
                for c in range(_CPW)
            ]
            writes = []
            for c in range(_CPW):
                gathers[c].wait()
                writes.append(pltpu.async_copy(
                    rows_v.at[c],
                    out_hbm.at[pl.ds(base + c * _CHUNK, _CHUNK)],
                    wsem))
            for wr in writes:
                wr.wait()

        @pl.when(wid == tail_wid)
        def _tail():
            for c in range(tail_count):
                pltpu.sync_copy(
                    idx_hbm.at[pl.ds(base + c * _CHUNK, _CHUNK)],
                    idx_v.at[pl.ds(c * _CHUNK, _CHUNK)])
                pltpu.async_copy(
                    table_hbm.at[idx_v.at[pl.ds(c * _CHUNK, _CHUNK)]],
                    rows_v.at[c], gsem).wait()
                pltpu.sync_copy(
                    rows_v.at[c],
                    out_hbm.at[pl.ds(base + c * _CHUNK, _CHUNK)])

    return gk


def kernel(x, selected_index, support, w):
    n = x.shape[0]
    s = _spmm(support[0], x, w)
    for i in range(1, support.shape[0]):
        s = s + _spmm(support[i], x, w)
    return _make_gather(n, w.shape[1])(s, selected_index.astype(jnp.int32))
